# Initial kernel scaffold; baseline (speedup 1.0000x reference)
import jax
import jax.numpy as jnp
from jax.experimental import pallas as pl


def kernel(x, c, edge_index, W_ada, b_ada, Wq, bq, Wk, bk, Wv, bv, Wo, bo, W1, b1, W2, b2):
    raise NotImplementedError("write your pallas kernel here")


# speedup vs baseline: 14.3898x; 14.3898x over previous
import functools

import jax
import jax.numpy as jnp
from jax import lax
from jax.experimental import pallas as pl
from jax.experimental.pallas import tpu as pltpu
from jax.experimental.pallas import tpu_sc as plsc

N = 10000
E = 320000
D = 128
H = 8
DH = 16
MLP = 512
SCALE = DH ** -0.5

_TAKE_DNUMS = lax.GatherDimensionNumbers(
    offset_dims=(), collapsed_slice_dims=(0,), start_index_map=(0,))


def _lane_take(t, idx):
    return lax.gather(t, idx[:, None], _TAKE_DNUMS, (1,),
                      mode=lax.GatherScatterMode.PROMISE_IN_BOUNDS)

NC = 2
NS = 16
NW = NC * NS
EPW = E // NW
W = 40
NWIN = EPW // W
NPAD = 10240
RPS = NPAD // NS
DPAD = NPAD // 8
DPS = DPAD // NS
CHUNKS = ((0, 0), (16, 0), (24, 8))


def _ln(x, eps=1e-6):
    m = jnp.mean(x, axis=-1, keepdims=True)
    v = jnp.var(x, axis=-1, keepdims=True)
    return (x - m) / jnp.sqrt(v + eps)


def _silu(x):
    return x * jax.nn.sigmoid(x)


def _gelu_tanh(x):
    return 0.5 * x * (1.0 + jnp.tanh(jnp.sqrt(2.0 / jnp.pi) * (x + 0.044715 * x ** 3)))


def _pre_body(x_ref, c_ref, wada_ref, bada_ref, wq_ref, bq_ref, wk_ref, bk_ref,
              wv_ref, bv_ref, q_ref, k_ref, v_ref, gmsa_ref, smlp_ref,
              hmlp_ref, gmlp_ref):
    sc = _silu(c_ref[...])
    ada = jnp.dot(sc, wada_ref[...], preferred_element_type=jnp.float32)
    ada = ada + bada_ref[...]
    shift_msa = ada[:, 0 * D:1 * D]
    scale_msa = ada[:, 1 * D:2 * D]
    gmsa_ref[...] = ada[:, 2 * D:3 * D]
    hmlp_ref[...] = ada[:, 3 * D:4 * D]
    smlp_ref[...] = ada[:, 4 * D:5 * D]
    gmlp_ref[...] = ada[:, 5 * D:6 * D]
    h = _ln(x_ref[...]) * (1.0 + scale_msa) + shift_msa
    q_ref[...] = jnp.dot(h, wq_ref[...], preferred_element_type=jnp.float32) + bq_ref[...]
    k_ref[...] = jnp.dot(h, wk_ref[...], preferred_element_type=jnp.float32) + bk_ref[...]
    v_ref[...] = jnp.dot(h, wv_ref[...], preferred_element_type=jnp.float32) + bv_ref[...]


def _pre_call(x, c, W_ada, b_ada, Wq, bq, Wk, bk, Wv, bv):
    R = 1000
    grid = (N // R,)
    row = pl.BlockSpec((R, D), lambda i: (i, 0))
    wide = pl.BlockSpec((D, 6 * D), lambda i: (0, 0))
    sq = pl.BlockSpec((D, D), lambda i: (0, 0))
    b6 = pl.BlockSpec((1, 6 * D), lambda i: (0, 0))
    b1 = pl.BlockSpec((1, D), lambda i: (0, 0))
    out = jax.ShapeDtypeStruct((N, D), jnp.float32)
    return pl.pallas_call(
        _pre_body,
        grid=grid,
        in_specs=[row, row, wide, b6, sq, b1, sq, b1, sq, b1],
        out_specs=[row] * 7,
        out_shape=[out] * 7,
    )(x, c, W_ada, b_ada.reshape(1, -1), Wq, bq.reshape(1, -1),
      Wk, bk.reshape(1, -1), Wv, bv.reshape(1, -1))


def _edge_body(q_hbm, k_hbm, v_hbm, src_hbm, dst_hbm, z128_hbm,
               agg_hbm, den_hbm, idx_src, idx_dst, idx8, qr, kr, vr, denrow,
               agg_sh, den_sh, sem):
    cid = lax.axis_index("c")
    sid = lax.axis_index("s")
    wid = sid * NC + cid

    pltpu.sync_copy(z128_hbm, agg_sh.at[pl.ds(sid * RPS, RPS)])
    pltpu.sync_copy(z128_hbm.at[pl.ds(0, DPS)], den_sh.at[pl.ds(sid * DPS, DPS)])
    plsc.subcore_barrier()

    ebase = wid * EPW
    lanes = lax.iota(jnp.int32, 16)
    lanesf = lanes.astype(jnp.float32)
    zv = lanesf * 0.0
    hmask = [jnp.maximum(1.0 - (lanesf - float(h)) * (lanesf - float(h)), 0.0)
             for h in range(H)]

    def window(w, _):
        off = ebase + w * W
        pltpu.async_copy(src_hbm.at[pl.ds(off, W)], idx_src, sem).wait()
        pltpu.async_copy(dst_hbm.at[pl.ds(off, W)], idx_dst, sem).wait()
        pltpu.async_copy(q_hbm.at[idx_dst], qr, sem).wait()
        pltpu.async_copy(k_hbm.at[idx_src], kr, sem).wait()
        pltpu.async_copy(v_hbm.at[idx_src], vr, sem).wait()

        for cstart, estart in CHUNKS:
            dchunk = idx_dst[pl.ds(cstart, 16)]
            idx8[pl.ds(cstart, 16)] = lax.shift_right_logical(dchunk, 3)

            def edge(e, _):
                ei = cstart + e
                dsplat = _lane_take(dchunk, jnp.full((16,), e, jnp.int32))
                m8 = jnp.bitwise_and(dsplat, 7)
                dv = zv
                for h in range(H):
                    qv = qr[ei, pl.ds(h * DH, DH)]
                    kv = kr[ei, pl.ds(h * DH, DH)]
                    t = qv * kv
                    for st in (8, 4, 2, 1):
                        t = t + _lane_take(t, lanes ^ st)
                    a = jnp.exp(t * SCALE)
                    vr[ei, pl.ds(h * DH, DH)] = vr[ei, pl.ds(h * DH, DH)] * a
                    dv = dv + a * hmask[h]
                for j in range(8):
                    df = (m8 - j).astype(jnp.float32)
                    denrow[ei, pl.ds(j * DH, DH)] = dv * jnp.maximum(
                        1.0 - df * df, 0.0)
                return 0

            lax.fori_loop(estart, 16, edge, 0)

        pltpu.sync_copy(vr, agg_sh.at[idx_dst], add=True)
        pltpu.sync_copy(denrow, den_sh.at[idx8], add=True)
        return 0

    lax.fori_loop(0, NWIN, window, 0)
    plsc.subcore_barrier()

    pltpu.sync_copy(agg_sh.at[pl.ds(sid * RPS, RPS)],
                    agg_hbm.at[cid, pl.ds(sid * RPS, RPS)])
    pltpu.sync_copy(den_sh.at[pl.ds(sid * DPS, DPS)],
                    den_hbm.at[cid, pl.ds(sid * DPS, DPS)])


def _edge_call(q, k, v, src, dst):
    z128 = jnp.zeros((RPS, D), jnp.float32)
    mesh = plsc.VectorSubcoreMesh(core_axis_name="c", subcore_axis_name="s")
    fn = pl.kernel(
        _edge_body,
        out_type=[
            jax.ShapeDtypeStruct((NC, NPAD, D), jnp.float32),
            jax.ShapeDtypeStruct((NC, DPAD, D), jnp.float32),
        ],
        mesh=mesh,
        scratch_types=[
            pltpu.VMEM((W,), jnp.int32),
            pltpu.VMEM((W,), jnp.int32),
            pltpu.VMEM((W,), jnp.int32),
            pltpu.VMEM((W, D), jnp.float32),
            pltpu.VMEM((W, D), jnp.float32),
            pltpu.VMEM((W, D), jnp.float32),
            pltpu.VMEM((W, D), jnp.float32),
            pltpu.VMEM_SHARED((NPAD, D), jnp.float32),
            pltpu.VMEM_SHARED((DPAD, D), jnp.float32),
            pltpu.SemaphoreType.DMA,
        ],
    )
    return fn(q, k, v, src, dst, z128)


def _post_body(x_ref, agg0_ref, agg1_ref, den0_ref, den1_ref, gmsa_ref,
               smlp_ref, hmlp_ref, gmlp_ref, wo_ref, bo_ref, w1_ref, b1_ref,
               w2_ref, b2_ref, out_ref):
    R = agg0_ref.shape[0]
    agg = agg0_ref[...] + agg1_ref[...]
    den = (den0_ref[...] + den1_ref[...])[:, :H]
    aggn = agg.reshape(R, H, DH) / (den.reshape(R, H, 1) + 1e-16)
    attn_out = jnp.dot(aggn.reshape(R, D), wo_ref[...],
                       preferred_element_type=jnp.float32) + bo_ref[...]
    x1 = x_ref[...] + gmsa_ref[...] * attn_out
    h2 = _ln(x1) * (1.0 + smlp_ref[...]) + hmlp_ref[...]
    t = jnp.dot(h2, w1_ref[...], preferred_element_type=jnp.float32) + b1_ref[...]
    mlp = jnp.dot(_gelu_tanh(t), w2_ref[...],
                  preferred_element_type=jnp.float32) + b2_ref[...]
    out_ref[...] = x1 + gmlp_ref[...] * mlp


def _post_call(x, agg, den, gmsa, smlp, hmlp, gmlp, Wo, bo, W1, b1, W2, b2):
    R = 1000
    grid = (N // R,)
    row = pl.BlockSpec((R, D), lambda i: (i, 0))
    row16 = pl.BlockSpec((R, 16), lambda i: (i, 0))
    sq = pl.BlockSpec((D, D), lambda i: (0, 0))
    wmlp1 = pl.BlockSpec((D, MLP), lambda i: (0, 0))
    wmlp2 = pl.BlockSpec((MLP, D), lambda i: (0, 0))
    b1s = pl.BlockSpec((1, D), lambda i: (0, 0))
    bm = pl.BlockSpec((1, MLP), lambda i: (0, 0))
    return pl.pallas_call(
        _post_body,
        grid=grid,
        in_specs=[row, row, row, row16, row16, row, row, row, row,
                  sq, b1s, wmlp1, bm, wmlp2, b1s],
        out_specs=row,
        out_shape=jax.ShapeDtypeStruct((N, D), jnp.float32),
    )(x, agg[0], agg[1], den[0], den[1], gmsa, smlp, hmlp, gmlp,
      Wo, bo.reshape(1, -1), W1, b1.reshape(1, -1), W2, b2.reshape(1, -1))


def kernel(x, c, edge_index, W_ada, b_ada, Wq, bq, Wk, bk, Wv, bv, Wo, bo,
           W1, b1, W2, b2):
    q, k, v, gmsa, smlp, hmlp, gmlp = _pre_call(
        x, c, W_ada, b_ada, Wq, bq, Wk, bk, Wv, bv)
    src = edge_index[0]
    dst = edge_index[1]
    agg, den = _edge_call(q, k, v, src, dst)
    agg = agg[:, :N]
    den = den.reshape(NC, NPAD, 16)[:, :N]
    return _post_call(x, agg, den, gmsa, smlp, hmlp, gmlp, Wo, bo, W1, b1, W2, b2)


# speedup vs baseline: 36.0739x; 2.5069x over previous
import functools

import jax
import jax.numpy as jnp
from jax import lax
from jax.experimental import pallas as pl
from jax.experimental.pallas import tpu as pltpu
from jax.experimental.pallas import tpu_sc as plsc

N = 10000
E = 320000
D = 128
H = 8
DH = 16
MLP = 512
SCALE = DH ** -0.5

_TAKE_DNUMS = lax.GatherDimensionNumbers(
    offset_dims=(), collapsed_slice_dims=(0,), start_index_map=(0,))


def _lane_take(t, idx):
    return lax.gather(t, idx[:, None], _TAKE_DNUMS, (1,),
                      mode=lax.GatherScatterMode.PROMISE_IN_BOUNDS)

NC = 2
NS = 16
NW = NC * NS
EPW = E // NW
W = 40
NWIN = EPW // W
NPAD = 10240
RPS = NPAD // NS
DPAD = NPAD // 8
DPS = DPAD // NS
CHUNKS = ((0, 0), (16, 0), (24, 8))


def _ln(x, eps=1e-6):
    m = jnp.mean(x, axis=-1, keepdims=True)
    v = jnp.var(x, axis=-1, keepdims=True)
    return (x - m) / jnp.sqrt(v + eps)


def _silu(x):
    return x * jax.nn.sigmoid(x)


def _gelu_tanh(x):
    return 0.5 * x * (1.0 + jnp.tanh(jnp.sqrt(2.0 / jnp.pi) * (x + 0.044715 * x ** 3)))


def _pre_body(x_ref, c_ref, wada_ref, bada_ref, wq_ref, bq_ref, wk_ref, bk_ref,
              wv_ref, bv_ref, q_ref, k_ref, v_ref, gmsa_ref, smlp_ref,
              hmlp_ref, gmlp_ref):
    sc = _silu(c_ref[...])
    ada = jnp.dot(sc, wada_ref[...], preferred_element_type=jnp.float32)
    ada = ada + bada_ref[...]
    shift_msa = ada[:, 0 * D:1 * D]
    scale_msa = ada[:, 1 * D:2 * D]
    gmsa_ref[...] = ada[:, 2 * D:3 * D]
    hmlp_ref[...] = ada[:, 3 * D:4 * D]
    smlp_ref[...] = ada[:, 4 * D:5 * D]
    gmlp_ref[...] = ada[:, 5 * D:6 * D]
    h = _ln(x_ref[...]) * (1.0 + scale_msa) + shift_msa
    q_ref[...] = jnp.dot(h, wq_ref[...], preferred_element_type=jnp.float32) + bq_ref[...]
    k_ref[...] = jnp.dot(h, wk_ref[...], preferred_element_type=jnp.float32) + bk_ref[...]
    v_ref[...] = jnp.dot(h, wv_ref[...], preferred_element_type=jnp.float32) + bv_ref[...]


def _pre_call(x, c, W_ada, b_ada, Wq, bq, Wk, bk, Wv, bv):
    R = 1000
    grid = (N // R,)
    row = pl.BlockSpec((R, D), lambda i: (i, 0))
    wide = pl.BlockSpec((D, 6 * D), lambda i: (0, 0))
    sq = pl.BlockSpec((D, D), lambda i: (0, 0))
    b6 = pl.BlockSpec((1, 6 * D), lambda i: (0, 0))
    b1 = pl.BlockSpec((1, D), lambda i: (0, 0))
    out = jax.ShapeDtypeStruct((N, D), jnp.float32)
    return pl.pallas_call(
        _pre_body,
        grid=grid,
        in_specs=[row, row, wide, b6, sq, b1, sq, b1, sq, b1],
        out_specs=[row] * 7,
        out_shape=[out] * 7,
    )(x, c, W_ada, b_ada.reshape(1, -1), Wq, bq.reshape(1, -1),
      Wk, bk.reshape(1, -1), Wv, bv.reshape(1, -1))


def _edge_body(q_hbm, k_hbm, v_hbm, src_hbm, dst_hbm, z128_hbm,
               agg_hbm, den_hbm, idx_src, idx_dst, idx8, qr, kr, vr, denrow,
               agg_sh, den_sh, sem):
    cid = lax.axis_index("c")
    sid = lax.axis_index("s")
    wid = sid * NC + cid

    pltpu.sync_copy(z128_hbm, agg_sh.at[pl.ds(sid * RPS, RPS)])
    pltpu.sync_copy(z128_hbm.at[pl.ds(0, DPS)], den_sh.at[pl.ds(sid * DPS, DPS)])
    pltpu.sync_copy(z128_hbm.at[pl.ds(0, W)], denrow)
    plsc.subcore_barrier()

    ebase = wid * EPW
    lanes = lax.iota(jnp.int32, 16)
    lanesf = lanes.astype(jnp.float32)
    zv = lanesf * 0.0
    hmask = [jnp.maximum(1.0 - (lanesf - float(h)) * (lanesf - float(h)), 0.0)
             for h in range(H)]

    def window(w, _):
        off = ebase + w * W
        c1 = pltpu.async_copy(src_hbm.at[pl.ds(off, W)], idx_src, sem)
        c2 = pltpu.async_copy(dst_hbm.at[pl.ds(off, W)], idx_dst, sem)
        c1.wait()
        c2.wait()
        c3 = pltpu.async_copy(q_hbm.at[idx_dst], qr, sem)
        c4 = pltpu.async_copy(k_hbm.at[idx_src], kr, sem)
        c5 = pltpu.async_copy(v_hbm.at[idx_src], vr, sem)
        c3.wait()
        c4.wait()
        c5.wait()

        for cstart, estart in CHUNKS:
            dchunk = idx_dst[pl.ds(cstart, 16)]
            idx8[pl.ds(cstart, 16)] = lax.shift_right_logical(dchunk, 3)

            def edge(e, _):
                ei = cstart + e
                m8s = jnp.bitwise_and(idx_dst[pl.ds(ei, 1)][0], 7)
                dv = zv
                for h in range(H):
                    qv = qr[ei, pl.ds(h * DH, DH)]
                    kv = kr[ei, pl.ds(h * DH, DH)]
                    t = qv * kv
                    for st in (8, 4, 2, 1):
                        t = t + _lane_take(t, lanes ^ st)
                    a = jnp.exp(t * SCALE)
                    vr[ei, pl.ds(h * DH, DH)] = vr[ei, pl.ds(h * DH, DH)] * a
                    dv = dv + a * hmask[h]
                denrow[ei, pl.ds(m8s * DH, DH)] = dv
                return 0

            lax.fori_loop(estart, 16, edge, 0)

        pltpu.sync_copy(vr, agg_sh.at[idx_dst], add=True)
        pltpu.sync_copy(denrow, den_sh.at[idx8], add=True)

        def clr(e, _):
            m8c = jnp.bitwise_and(idx_dst[pl.ds(e, 1)][0], 7)
            denrow[e, pl.ds(m8c * DH, DH)] = zv
            return 0

        lax.fori_loop(0, W, clr, 0)
        return 0

    lax.fori_loop(0, NWIN, window, 0)
    plsc.subcore_barrier()

    pltpu.sync_copy(agg_sh.at[pl.ds(sid * RPS, RPS)],
                    agg_hbm.at[cid, pl.ds(sid * RPS, RPS)])
    pltpu.sync_copy(den_sh.at[pl.ds(sid * DPS, DPS)],
                    den_hbm.at[cid, pl.ds(sid * DPS, DPS)])


def _edge_call(q, k, v, src, dst):
    z128 = jnp.zeros((RPS, D), jnp.float32)
    mesh = plsc.VectorSubcoreMesh(core_axis_name="c", subcore_axis_name="s")
    fn = pl.kernel(
        _edge_body,
        out_type=[
            jax.ShapeDtypeStruct((NC, NPAD, D), jnp.float32),
            jax.ShapeDtypeStruct((NC, DPAD, D), jnp.float32),
        ],
        mesh=mesh,
        scratch_types=[
            pltpu.VMEM((W,), jnp.int32),
            pltpu.VMEM((W,), jnp.int32),
            pltpu.VMEM((W,), jnp.int32),
            pltpu.VMEM((W, D), jnp.float32),
            pltpu.VMEM((W, D), jnp.float32),
            pltpu.VMEM((W, D), jnp.float32),
            pltpu.VMEM((W, D), jnp.float32),
            pltpu.VMEM_SHARED((NPAD, D), jnp.float32),
            pltpu.VMEM_SHARED((DPAD, D), jnp.float32),
            pltpu.SemaphoreType.DMA,
        ],
    )
    return fn(q, k, v, src, dst, z128)


def _post_body(x_ref, agg0_ref, agg1_ref, den0_ref, den1_ref, gmsa_ref,
               smlp_ref, hmlp_ref, gmlp_ref, wo_ref, bo_ref, w1_ref, b1_ref,
               w2_ref, b2_ref, out_ref):
    R = agg0_ref.shape[0]
    agg = agg0_ref[...] + agg1_ref[...]
    den = (den0_ref[...] + den1_ref[...])[:, :H]
    aggn = agg.reshape(R, H, DH) / (den.reshape(R, H, 1) + 1e-16)
    attn_out = jnp.dot(aggn.reshape(R, D), wo_ref[...],
                       preferred_element_type=jnp.float32) + bo_ref[...]
    x1 = x_ref[...] + gmsa_ref[...] * attn_out
    h2 = _ln(x1) * (1.0 + smlp_ref[...]) + hmlp_ref[...]
    t = jnp.dot(h2, w1_ref[...], preferred_element_type=jnp.float32) + b1_ref[...]
    mlp = jnp.dot(_gelu_tanh(t), w2_ref[...],
                  preferred_element_type=jnp.float32) + b2_ref[...]
    out_ref[...] = x1 + gmlp_ref[...] * mlp


def _post_call(x, agg, den, gmsa, smlp, hmlp, gmlp, Wo, bo, W1, b1, W2, b2):
    R = 1000
    grid = (N // R,)
    row = pl.BlockSpec((R, D), lambda i: (i, 0))
    row16 = pl.BlockSpec((R, 16), lambda i: (i, 0))
    sq = pl.BlockSpec((D, D), lambda i: (0, 0))
    wmlp1 = pl.BlockSpec((D, MLP), lambda i: (0, 0))
    wmlp2 = pl.BlockSpec((MLP, D), lambda i: (0, 0))
    b1s = pl.BlockSpec((1, D), lambda i: (0, 0))
    bm = pl.BlockSpec((1, MLP), lambda i: (0, 0))
    return pl.pallas_call(
        _post_body,
        grid=grid,
        in_specs=[row, row, row, row16, row16, row, row, row, row,
                  sq, b1s, wmlp1, bm, wmlp2, b1s],
        out_specs=row,
        out_shape=jax.ShapeDtypeStruct((N, D), jnp.float32),
    )(x, agg[0], agg[1], den[0], den[1], gmsa, smlp, hmlp, gmlp,
      Wo, bo.reshape(1, -1), W1, b1.reshape(1, -1), W2, b2.reshape(1, -1))


def kernel(x, c, edge_index, W_ada, b_ada, Wq, bq, Wk, bk, Wv, bv, Wo, bo,
           W1, b1, W2, b2):
    q, k, v, gmsa, smlp, hmlp, gmlp = _pre_call(
        x, c, W_ada, b_ada, Wq, bq, Wk, bk, Wv, bv)
    src = edge_index[0]
    dst = edge_index[1]
    agg, den = _edge_call(q, k, v, src, dst)
    agg = agg[:, :N]
    den = den.reshape(NC, NPAD, 16)[:, :N]
    return _post_call(x, agg, den, gmsa, smlp, hmlp, gmlp, Wo, bo, W1, b1, W2, b2)


# speedup vs baseline: 48.5345x; 1.3454x over previous
import functools

import jax
import jax.numpy as jnp
from jax import lax
from jax.experimental import pallas as pl
from jax.experimental.pallas import tpu as pltpu
from jax.experimental.pallas import tpu_sc as plsc

N = 10000
E = 320000
D = 128
H = 8
DH = 16
MLP = 512
SCALE = DH ** -0.5

_TAKE_DNUMS = lax.GatherDimensionNumbers(
    offset_dims=(), collapsed_slice_dims=(0,), start_index_map=(0,))


def _lane_take(t, idx):
    return lax.gather(t, idx[:, None], _TAKE_DNUMS, (1,),
                      mode=lax.GatherScatterMode.PROMISE_IN_BOUNDS)

NC = 2
NS = 16
NW = NC * NS
EPW = E // NW
W = 40
NWIN = EPW // W
NPAD = 10240
RPS = NPAD // NS
DPAD = NPAD // 8
DPS = DPAD // NS
CHUNKS = ((0, 0), (16, 0), (24, 8))


def _ln(x, eps=1e-6):
    m = jnp.mean(x, axis=-1, keepdims=True)
    v = jnp.var(x, axis=-1, keepdims=True)
    return (x - m) / jnp.sqrt(v + eps)


def _silu(x):
    return x * jax.nn.sigmoid(x)


def _gelu_tanh(x):
    return 0.5 * x * (1.0 + jnp.tanh(jnp.sqrt(2.0 / jnp.pi) * (x + 0.044715 * x ** 3)))


def _pre_body(x_ref, c_ref, wada_ref, bada_ref, wq_ref, bq_ref, wk_ref, bk_ref,
              wv_ref, bv_ref, q_ref, k_ref, v_ref, gmsa_ref, smlp_ref,
              hmlp_ref, gmlp_ref):
    sc = _silu(c_ref[...])
    ada = jnp.dot(sc, wada_ref[...], preferred_element_type=jnp.float32)
    ada = ada + bada_ref[...]
    shift_msa = ada[:, 0 * D:1 * D]
    scale_msa = ada[:, 1 * D:2 * D]
    gmsa_ref[...] = ada[:, 2 * D:3 * D]
    hmlp_ref[...] = ada[:, 3 * D:4 * D]
    smlp_ref[...] = ada[:, 4 * D:5 * D]
    gmlp_ref[...] = ada[:, 5 * D:6 * D]
    h = _ln(x_ref[...]) * (1.0 + scale_msa) + shift_msa
    q_ref[...] = jnp.dot(h, wq_ref[...], preferred_element_type=jnp.float32) + bq_ref[...]
    k_ref[...] = jnp.dot(h, wk_ref[...], preferred_element_type=jnp.float32) + bk_ref[...]
    v_ref[...] = jnp.dot(h, wv_ref[...], preferred_element_type=jnp.float32) + bv_ref[...]


def _pre_call(x, c, W_ada, b_ada, Wq, bq, Wk, bk, Wv, bv):
    R = 1000
    grid = (N // R,)
    row = pl.BlockSpec((R, D), lambda i: (i, 0))
    wide = pl.BlockSpec((D, 6 * D), lambda i: (0, 0))
    sq = pl.BlockSpec((D, D), lambda i: (0, 0))
    b6 = pl.BlockSpec((1, 6 * D), lambda i: (0, 0))
    b1 = pl.BlockSpec((1, D), lambda i: (0, 0))
    out = jax.ShapeDtypeStruct((N, D), jnp.float32)
    return pl.pallas_call(
        _pre_body,
        grid=grid,
        in_specs=[row, row, wide, b6, sq, b1, sq, b1, sq, b1],
        out_specs=[row] * 7,
        out_shape=[out] * 7,
    )(x, c, W_ada, b_ada.reshape(1, -1), Wq, bq.reshape(1, -1),
      Wk, bk.reshape(1, -1), Wv, bv.reshape(1, -1))


def _edge_body(q_hbm, k_hbm, v_hbm, src_hbm, dst_hbm, z128_hbm,
               agg_hbm, den_hbm, is0, id0, is1, id1, idx8,
               q0, k0, v0, q1, k1, v1, denrow,
               agg_sh, den_sh, sem0, sem1):
    cid = lax.axis_index("c")
    sid = lax.axis_index("s")
    wid = sid * NC + cid
    bufs = ((is0, id0, q0, k0, v0, sem0), (is1, id1, q1, k1, v1, sem1))

    pltpu.sync_copy(z128_hbm, agg_sh.at[pl.ds(sid * RPS, RPS)])
    pltpu.sync_copy(z128_hbm.at[pl.ds(0, DPS)], den_sh.at[pl.ds(sid * DPS, DPS)])
    pltpu.sync_copy(z128_hbm.at[pl.ds(0, W)], denrow)
    plsc.subcore_barrier()

    ebase = wid * EPW
    lanes = lax.iota(jnp.int32, 16)
    lanesf = lanes.astype(jnp.float32)
    zv = lanesf * 0.0
    hmask = [jnp.maximum(1.0 - (lanesf - float(h)) * (lanesf - float(h)), 0.0)
             for h in range(H)]

    def start(w, b):
        isx, idx, qr, kr, vr, sem = bufs[b]
        off = ebase + w * W
        c1 = pltpu.async_copy(src_hbm.at[pl.ds(off, W)], isx, sem)
        c2 = pltpu.async_copy(dst_hbm.at[pl.ds(off, W)], idx, sem)
        c1.wait()
        c2.wait()
        pltpu.async_copy(q_hbm.at[idx], qr, sem)
        pltpu.async_copy(k_hbm.at[isx], kr, sem)
        pltpu.async_copy(v_hbm.at[isx], vr, sem)

    def drain(b):
        isx, idx, qr, kr, vr, sem = bufs[b]
        pltpu.make_async_copy(q_hbm.at[idx], qr, sem).wait()
        pltpu.make_async_copy(k_hbm.at[isx], kr, sem).wait()
        pltpu.make_async_copy(v_hbm.at[isx], vr, sem).wait()

    def compute(w, b):
        isx, idx_dst, qr, kr, vr, sem = bufs[b]
        drain(b)

        for cstart, estart in CHUNKS:
            dchunk = idx_dst[pl.ds(cstart, 16)]
            idx8[pl.ds(cstart, 16)] = lax.shift_right_logical(dchunk, 3)

            def edge(e, _):
                ei = cstart + e
                m8s = jnp.bitwise_and(idx_dst[pl.ds(ei, 1)][0], 7)
                dv = zv
                for h in range(H):
                    qv = qr[ei, pl.ds(h * DH, DH)]
                    kv = kr[ei, pl.ds(h * DH, DH)]
                    t = qv * kv
                    for st in (8, 4, 2, 1):
                        t = t + _lane_take(t, lanes ^ st)
                    a = jnp.exp(t * SCALE)
                    vr[ei, pl.ds(h * DH, DH)] = vr[ei, pl.ds(h * DH, DH)] * a
                    dv = dv + a * hmask[h]
                denrow[ei, pl.ds(m8s * DH, DH)] = dv
                return 0

            lax.fori_loop(estart, 16, edge, 0)

        pltpu.sync_copy(vr, agg_sh.at[idx_dst], add=True)
        pltpu.sync_copy(denrow, den_sh.at[idx8], add=True)

        def clr(e, _):
            m8c = jnp.bitwise_and(idx_dst[pl.ds(e, 1)][0], 7)
            denrow[e, pl.ds(m8c * DH, DH)] = zv
            return 0

        lax.fori_loop(0, W, clr, 0)

    start(0, 0)

    def outer(t, _):
        for b in range(2):
            w = t * 2 + b
            start(jnp.minimum(w + 1, NWIN - 1), 1 - b)
            compute(w, b)
        return 0

    lax.fori_loop(0, NWIN // 2, outer, 0)
    drain(0)
    plsc.subcore_barrier()

    pltpu.sync_copy(agg_sh.at[pl.ds(sid * RPS, RPS)],
                    agg_hbm.at[cid, pl.ds(sid * RPS, RPS)])
    pltpu.sync_copy(den_sh.at[pl.ds(sid * DPS, DPS)],
                    den_hbm.at[cid, pl.ds(sid * DPS, DPS)])


def _edge_call(q, k, v, src, dst):
    z128 = jnp.zeros((RPS, D), jnp.float32)
    mesh = plsc.VectorSubcoreMesh(core_axis_name="c", subcore_axis_name="s")
    fn = pl.kernel(
        _edge_body,
        out_type=[
            jax.ShapeDtypeStruct((NC, NPAD, D), jnp.float32),
            jax.ShapeDtypeStruct((NC, DPAD, D), jnp.float32),
        ],
        mesh=mesh,
        scratch_types=[
            pltpu.VMEM((W,), jnp.int32),
            pltpu.VMEM((W,), jnp.int32),
            pltpu.VMEM((W,), jnp.int32),
            pltpu.VMEM((W,), jnp.int32),
            pltpu.VMEM((W,), jnp.int32),
            pltpu.VMEM((W, D), jnp.float32),
            pltpu.VMEM((W, D), jnp.float32),
            pltpu.VMEM((W, D), jnp.float32),
            pltpu.VMEM((W, D), jnp.float32),
            pltpu.VMEM((W, D), jnp.float32),
            pltpu.VMEM((W, D), jnp.float32),
            pltpu.VMEM((W, D), jnp.float32),
            pltpu.VMEM_SHARED((NPAD, D), jnp.float32),
            pltpu.VMEM_SHARED((DPAD, D), jnp.float32),
            pltpu.SemaphoreType.DMA,
            pltpu.SemaphoreType.DMA,
        ],
    )
    return fn(q, k, v, src, dst, z128)


def _post_body(x_ref, agg0_ref, agg1_ref, den0_ref, den1_ref, gmsa_ref,
               smlp_ref, hmlp_ref, gmlp_ref, wo_ref, bo_ref, w1_ref, b1_ref,
               w2_ref, b2_ref, out_ref):
    R = agg0_ref.shape[0]
    agg = agg0_ref[...] + agg1_ref[...]
    den = (den0_ref[...] + den1_ref[...])[:, :H]
    aggn = agg.reshape(R, H, DH) / (den.reshape(R, H, 1) + 1e-16)
    attn_out = jnp.dot(aggn.reshape(R, D), wo_ref[...],
                       preferred_element_type=jnp.float32) + bo_ref[...]
    x1 = x_ref[...] + gmsa_ref[...] * attn_out
    h2 = _ln(x1) * (1.0 + smlp_ref[...]) + hmlp_ref[...]
    t = jnp.dot(h2, w1_ref[...], preferred_element_type=jnp.float32) + b1_ref[...]
    mlp = jnp.dot(_gelu_tanh(t), w2_ref[...],
                  preferred_element_type=jnp.float32) + b2_ref[...]
    out_ref[...] = x1 + gmlp_ref[...] * mlp


def _post_call(x, agg, den, gmsa, smlp, hmlp, gmlp, Wo, bo, W1, b1, W2, b2):
    R = 1000
    grid = (N // R,)
    row = pl.BlockSpec((R, D), lambda i: (i, 0))
    row16 = pl.BlockSpec((R, 16), lambda i: (i, 0))
    sq = pl.BlockSpec((D, D), lambda i: (0, 0))
    wmlp1 = pl.BlockSpec((D, MLP), lambda i: (0, 0))
    wmlp2 = pl.BlockSpec((MLP, D), lambda i: (0, 0))
    b1s = pl.BlockSpec((1, D), lambda i: (0, 0))
    bm = pl.BlockSpec((1, MLP), lambda i: (0, 0))
    return pl.pallas_call(
        _post_body,
        grid=grid,
        in_specs=[row, row, row, row16, row16, row, row, row, row,
                  sq, b1s, wmlp1, bm, wmlp2, b1s],
        out_specs=row,
        out_shape=jax.ShapeDtypeStruct((N, D), jnp.float32),
    )(x, agg[0], agg[1], den[0], den[1], gmsa, smlp, hmlp, gmlp,
      Wo, bo.reshape(1, -1), W1, b1.reshape(1, -1), W2, b2.reshape(1, -1))


def kernel(x, c, edge_index, W_ada, b_ada, Wq, bq, Wk, bk, Wv, bv, Wo, bo,
           W1, b1, W2, b2):
    q, k, v, gmsa, smlp, hmlp, gmlp = _pre_call(
        x, c, W_ada, b_ada, Wq, bq, Wk, bk, Wv, bv)
    src = edge_index[0]
    dst = edge_index[1]
    agg, den = _edge_call(q, k, v, src, dst)
    agg = agg[:, :N]
    den = den.reshape(NC, NPAD, 16)[:, :N]
    return _post_call(x, agg, den, gmsa, smlp, hmlp, gmlp, Wo, bo, W1, b1, W2, b2)


# speedup vs baseline: 49.4704x; 1.0193x over previous
import functools

import jax
import jax.numpy as jnp
from jax import lax
from jax.experimental import pallas as pl
from jax.experimental.pallas import tpu as pltpu
from jax.experimental.pallas import tpu_sc as plsc

N = 10000
E = 320000
D = 128
H = 8
DH = 16
MLP = 512
SCALE = DH ** -0.5

_TAKE_DNUMS = lax.GatherDimensionNumbers(
    offset_dims=(), collapsed_slice_dims=(0,), start_index_map=(0,))


def _lane_take(t, idx):
    return lax.gather(t, idx[:, None], _TAKE_DNUMS, (1,),
                      mode=lax.GatherScatterMode.PROMISE_IN_BOUNDS)

NC = 2
NS = 16
NW = NC * NS
EPW = E // NW
W = 40
NWIN = EPW // W
NPAD = 10240
RPS = NPAD // NS
DPAD = NPAD // 8
DPS = DPAD // NS
CHUNKS = ((0, 0), (16, 0), (24, 8))


def _ln(x, eps=1e-6):
    m = jnp.mean(x, axis=-1, keepdims=True)
    v = jnp.var(x, axis=-1, keepdims=True)
    return (x - m) / jnp.sqrt(v + eps)


def _silu(x):
    return x * jax.nn.sigmoid(x)


def _gelu_tanh(x):
    return 0.5 * x * (1.0 + jnp.tanh(jnp.sqrt(2.0 / jnp.pi) * (x + 0.044715 * x ** 3)))


def _pre_body(x_ref, c_ref, wada_ref, bada_ref, wq_ref, bq_ref, wk_ref, bk_ref,
              wv_ref, bv_ref, q_ref, k_ref, v_ref, gmsa_ref, smlp_ref,
              hmlp_ref, gmlp_ref):
    sc = _silu(c_ref[...])
    ada = jnp.dot(sc, wada_ref[...], preferred_element_type=jnp.float32)
    ada = ada + bada_ref[...]
    shift_msa = ada[:, 0 * D:1 * D]
    scale_msa = ada[:, 1 * D:2 * D]
    gmsa_ref[...] = ada[:, 2 * D:3 * D]
    hmlp_ref[...] = ada[:, 3 * D:4 * D]
    smlp_ref[...] = ada[:, 4 * D:5 * D]
    gmlp_ref[...] = ada[:, 5 * D:6 * D]
    h = _ln(x_ref[...]) * (1.0 + scale_msa) + shift_msa
    q_ref[...] = jnp.dot(h, wq_ref[...], preferred_element_type=jnp.float32) + bq_ref[...]
    k_ref[...] = jnp.dot(h, wk_ref[...], preferred_element_type=jnp.float32) + bk_ref[...]
    v_ref[...] = jnp.dot(h, wv_ref[...], preferred_element_type=jnp.float32) + bv_ref[...]


def _pre_call(x, c, W_ada, b_ada, Wq, bq, Wk, bk, Wv, bv):
    R = 1000
    grid = (N // R,)
    row = pl.BlockSpec((R, D), lambda i: (i, 0))
    wide = pl.BlockSpec((D, 6 * D), lambda i: (0, 0))
    sq = pl.BlockSpec((D, D), lambda i: (0, 0))
    b6 = pl.BlockSpec((1, 6 * D), lambda i: (0, 0))
    b1 = pl.BlockSpec((1, D), lambda i: (0, 0))
    out = jax.ShapeDtypeStruct((N, D), jnp.float32)
    return pl.pallas_call(
        _pre_body,
        grid=grid,
        in_specs=[row, row, wide, b6, sq, b1, sq, b1, sq, b1],
        out_specs=[row] * 7,
        out_shape=[out] * 7,
    )(x, c, W_ada, b_ada.reshape(1, -1), Wq, bq.reshape(1, -1),
      Wk, bk.reshape(1, -1), Wv, bv.reshape(1, -1))


def _edge_body(q_hbm, k_hbm, v_hbm, src_hbm, dst_hbm, z128_hbm,
               agg_hbm, den_hbm, is0, id0, is1, id1, idx8,
               q0, k0, v0, q1, k1, v1, denrow,
               agg_sh, den_sh, sem0, sem1, ssem0, ssem1):
    cid = lax.axis_index("c")
    sid = lax.axis_index("s")
    wid = sid * NC + cid
    bufs = ((is0, id0, q0, k0, v0, sem0, ssem0),
            (is1, id1, q1, k1, v1, sem1, ssem1))

    pltpu.sync_copy(z128_hbm, agg_sh.at[pl.ds(sid * RPS, RPS)])
    pltpu.sync_copy(z128_hbm.at[pl.ds(0, DPS)], den_sh.at[pl.ds(sid * DPS, DPS)])
    pltpu.sync_copy(z128_hbm.at[pl.ds(0, W)], denrow)
    plsc.subcore_barrier()

    ebase = wid * EPW
    lanes = lax.iota(jnp.int32, 16)
    lanesf = lanes.astype(jnp.float32)
    zv = lanesf * 0.0
    hmask = [jnp.maximum(1.0 - (lanesf - float(h)) * (lanesf - float(h)), 0.0)
             for h in range(H)]

    def start(w, b, steady):
        isx, idx, qr, kr, vr, sem, ssem = bufs[b]
        if steady:
            pltpu.make_async_copy(vr, agg_sh.at[idx], ssem).wait()
        off = ebase + w * W
        c1 = pltpu.async_copy(src_hbm.at[pl.ds(off, W)], isx, sem)
        c2 = pltpu.async_copy(dst_hbm.at[pl.ds(off, W)], idx, sem)
        c1.wait()
        c2.wait()
        pltpu.async_copy(q_hbm.at[idx], qr, sem)
        pltpu.async_copy(k_hbm.at[isx], kr, sem)
        pltpu.async_copy(v_hbm.at[isx], vr, sem)

    def drain_scatters(b):
        isx, idx, qr, kr, vr, sem, ssem = bufs[b]
        pltpu.make_async_copy(vr, agg_sh.at[idx], ssem).wait()

    def compute(w, b):
        isx, idx_dst, qr, kr, vr, sem, ssem = bufs[b]
        pltpu.make_async_copy(q_hbm.at[idx_dst], qr, sem).wait()
        pltpu.make_async_copy(k_hbm.at[isx], kr, sem).wait()
        pltpu.make_async_copy(v_hbm.at[isx], vr, sem).wait()

        for cstart, estart in CHUNKS:
            dchunk = idx_dst[pl.ds(cstart, 16)]
            idx8[pl.ds(cstart, 16)] = lax.shift_right_logical(dchunk, 3)

            def edge(e, _):
                ei = cstart + e
                m8s = jnp.bitwise_and(idx_dst[pl.ds(ei, 1)][0], 7)
                dv = zv
                for h in range(H):
                    qv = qr[ei, pl.ds(h * DH, DH)]
                    kv = kr[ei, pl.ds(h * DH, DH)]
                    t = qv * kv
                    for st in (8, 4, 2, 1):
                        t = t + _lane_take(t, lanes ^ st)
                    a = jnp.exp(t * SCALE)
                    vr[ei, pl.ds(h * DH, DH)] = vr[ei, pl.ds(h * DH, DH)] * a
                    dv = dv + a * hmask[h]
                denrow[ei, pl.ds(m8s * DH, DH)] = dv
                return 0

            lax.fori_loop(estart, 16, edge, 0)

        pltpu.async_copy(vr, agg_sh.at[idx_dst], ssem, add=True)
        pltpu.sync_copy(denrow, den_sh.at[idx8], add=True)

        def clr(e, _):
            m8c = jnp.bitwise_and(idx_dst[pl.ds(e, 1)][0], 7)
            denrow[e, pl.ds(m8c * DH, DH)] = zv
            return 0

        lax.fori_loop(0, W, clr, 0)

    start(0, 0, steady=False)
    start(1, 1, steady=False)
    compute(0, 0)

    def outer(t, _):
        w = 2 * t + 1
        start(w + 1, 0, steady=True)
        compute(w, 1)
        start(w + 2, 1, steady=True)
        compute(w + 1, 0)
        return 0

    lax.fori_loop(0, (NWIN - 2) // 2, outer, 0)
    compute(NWIN - 1, 1)
    drain_scatters(0)
    drain_scatters(1)
    plsc.subcore_barrier()

    pltpu.sync_copy(agg_sh.at[pl.ds(sid * RPS, RPS)],
                    agg_hbm.at[cid, pl.ds(sid * RPS, RPS)])
    pltpu.sync_copy(den_sh.at[pl.ds(sid * DPS, DPS)],
                    den_hbm.at[cid, pl.ds(sid * DPS, DPS)])


def _edge_call(q, k, v, src, dst):
    z128 = jnp.zeros((RPS, D), jnp.float32)
    mesh = plsc.VectorSubcoreMesh(core_axis_name="c", subcore_axis_name="s")
    fn = pl.kernel(
        _edge_body,
        out_type=[
            jax.ShapeDtypeStruct((NC, NPAD, D), jnp.float32),
            jax.ShapeDtypeStruct((NC, DPAD, D), jnp.float32),
        ],
        mesh=mesh,
        scratch_types=(
            [pltpu.VMEM((W,), jnp.int32)] * 5
            + [pltpu.VMEM((W, D), jnp.float32)] * 7
            + [pltpu.VMEM_SHARED((NPAD, D), jnp.float32),
               pltpu.VMEM_SHARED((DPAD, D), jnp.float32)]
            + [pltpu.SemaphoreType.DMA] * 4
        ),
    )
    return fn(q, k, v, src, dst, z128)


def _post_body(x_ref, agg0_ref, agg1_ref, den0_ref, den1_ref, gmsa_ref,
               smlp_ref, hmlp_ref, gmlp_ref, wo_ref, bo_ref, w1_ref, b1_ref,
               w2_ref, b2_ref, out_ref):
    R = agg0_ref.shape[0]
    agg = agg0_ref[...] + agg1_ref[...]
    den = (den0_ref[...] + den1_ref[...])[:, :H]
    aggn = agg.reshape(R, H, DH) / (den.reshape(R, H, 1) + 1e-16)
    attn_out = jnp.dot(aggn.reshape(R, D), wo_ref[...],
                       preferred_element_type=jnp.float32) + bo_ref[...]
    x1 = x_ref[...] + gmsa_ref[...] * attn_out
    h2 = _ln(x1) * (1.0 + smlp_ref[...]) + hmlp_ref[...]
    t = jnp.dot(h2, w1_ref[...], preferred_element_type=jnp.float32) + b1_ref[...]
    mlp = jnp.dot(_gelu_tanh(t), w2_ref[...],
                  preferred_element_type=jnp.float32) + b2_ref[...]
    out_ref[...] = x1 + gmlp_ref[...] * mlp


def _post_call(x, agg, den, gmsa, smlp, hmlp, gmlp, Wo, bo, W1, b1, W2, b2):
    R = 1000
    grid = (N // R,)
    row = pl.BlockSpec((R, D), lambda i: (i, 0))
    row16 = pl.BlockSpec((R, 16), lambda i: (i, 0))
    sq = pl.BlockSpec((D, D), lambda i: (0, 0))
    wmlp1 = pl.BlockSpec((D, MLP), lambda i: (0, 0))
    wmlp2 = pl.BlockSpec((MLP, D), lambda i: (0, 0))
    b1s = pl.BlockSpec((1, D), lambda i: (0, 0))
    bm = pl.BlockSpec((1, MLP), lambda i: (0, 0))
    return pl.pallas_call(
        _post_body,
        grid=grid,
        in_specs=[row, row, row, row16, row16, row, row, row, row,
                  sq, b1s, wmlp1, bm, wmlp2, b1s],
        out_specs=row,
        out_shape=jax.ShapeDtypeStruct((N, D), jnp.float32),
    )(x, agg[0], agg[1], den[0], den[1], gmsa, smlp, hmlp, gmlp,
      Wo, bo.reshape(1, -1), W1, b1.reshape(1, -1), W2, b2.reshape(1, -1))


def kernel(x, c, edge_index, W_ada, b_ada, Wq, bq, Wk, bk, Wv, bv, Wo, bo,
           W1, b1, W2, b2):
    q, k, v, gmsa, smlp, hmlp, gmlp = _pre_call(
        x, c, W_ada, b_ada, Wq, bq, Wk, bk, Wv, bv)
    src = edge_index[0]
    dst = edge_index[1]
    agg, den = _edge_call(q, k, v, src, dst)
    agg = agg[:, :N]
    den = den.reshape(NC, NPAD, 16)[:, :N]
    return _post_call(x, agg, den, gmsa, smlp, hmlp, gmlp, Wo, bo, W1, b1, W2, b2)
